# Initial kernel scaffold; baseline (speedup 1.0000x reference)
import jax
import jax.numpy as jnp
from jax.experimental import pallas as pl


def kernel(adj, x, W0, a_src0, a_dst0, b0, W1, a_src1, a_dst1, b1, W2, a_src2, a_dst2, b2):
    raise NotImplementedError("write your pallas kernel here")


# speedup vs baseline: 6864.5027x; 6864.5027x over previous
import functools

import jax
import jax.numpy as jnp
from jax.experimental import pallas as pl

N = 2048
TD = 256


def _proj_body(x_ref, w_ref, asrc_ref, adst_ref, hh_ref, as_ref, ad_ref, *,
               heads, ch):
    h = jnp.dot(x_ref[...], w_ref[...], preferred_element_type=jnp.float32)
    as_ref[...] = jnp.dot(h, asrc_ref[...], preferred_element_type=jnp.float32)
    ad_ref[...] = jax.lax.dot_general(
        adst_ref[...], h, (((0,), (1,)), ((), ())),
        preferred_element_type=jnp.float32)
    for i in range(heads):
        hh_ref[i * N:(i + 1) * N, :] = h[:, i * ch:(i + 1) * ch]


def _attn_body(adj_ref, hh_ref, as_ref, ad_ref, b_ref, out_ref, *,
               heads, ch, self_loops, relu):
    j = pl.program_id(0)
    mask = adj_ref[...] != 0.0
    if self_loops:
        r = jax.lax.broadcasted_iota(jnp.int32, (N, TD), 0)
        c = jax.lax.broadcasted_iota(jnp.int32, (N, TD), 1) + j * TD
        mask = jnp.logical_or(mask, r == c)
    neg = jnp.float32(-1e30)
    acc = jnp.zeros((TD, ch), jnp.float32)
    for i in range(heads):
        e = as_ref[:, i:i + 1] + ad_ref[i:i + 1, :]
        e = jnp.where(e >= 0.0, e, e * jnp.float32(0.2))
        e = jnp.where(mask, e, neg)
        m = jnp.max(e, axis=0, keepdims=True)
        m = jnp.where(m > jnp.float32(-1e29), m, 0.0)
        p = jnp.exp(e - m)
        s = jnp.sum(p, axis=0, keepdims=True)
        alpha = p * (1.0 / (s + jnp.float32(1e-16)))
        hh = hh_ref[i * N:(i + 1) * N, :]
        acc = acc + jax.lax.dot_general(
            alpha, hh, (((0,), (0,)), ((), ())),
            preferred_element_type=jnp.float32)
    out = acc * jnp.float32(1.0 / heads) + b_ref[0:1, :]
    if relu:
        out = jnp.maximum(out, 0.0)
    out_ref[...] = out


def _block_diag(a, ch):
    hc = a.shape[0] * ch
    rows = jnp.arange(hc)
    return jnp.zeros((hc, 8), jnp.float32).at[rows, rows // ch].set(
        a.reshape(-1))


def _gat_layer(adj, x_in, W, a_src, a_dst, b, heads, ch, self_loops, relu):
    asrc = _block_diag(a_src, ch)
    adst = _block_diag(a_dst, ch)
    hh, as_p, ad_t = pl.pallas_call(
        functools.partial(_proj_body, heads=heads, ch=ch),
        out_shape=(jax.ShapeDtypeStruct((heads * N, ch), jnp.float32),
                   jax.ShapeDtypeStruct((N, 8), jnp.float32),
                   jax.ShapeDtypeStruct((8, N), jnp.float32)),
    )(x_in, W, asrc, adst)
    out = pl.pallas_call(
        functools.partial(_attn_body, heads=heads, ch=ch,
                          self_loops=self_loops, relu=relu),
        grid=(N // TD,),
        in_specs=[pl.BlockSpec((N, TD), lambda j: (0, j)),
                  pl.BlockSpec((heads * N, ch), lambda j: (0, 0)),
                  pl.BlockSpec((N, 8), lambda j: (0, 0)),
                  pl.BlockSpec((8, TD), lambda j: (0, j)),
                  pl.BlockSpec((1, ch), lambda j: (0, 0))],
        out_specs=pl.BlockSpec((TD, ch), lambda j: (j, 0)),
        out_shape=jax.ShapeDtypeStruct((N, ch), jnp.float32),
    )(adj, hh, as_p, ad_t, b.reshape(1, ch))
    return out


def kernel(adj, x, W0, a_src0, a_dst0, b0, W1, a_src1, a_dst1, b1,
           W2, a_src2, a_dst2, b2):
    h = _gat_layer(adj, x, W0, a_src0, a_dst0, b0, 4, 16, False, True)
    h = _gat_layer(adj, h, W1, a_src1, a_dst1, b1, 4, 16, True, True)
    return _gat_layer(adj, h, W2, a_src2, a_dst2, b2, 1, 32, True, False)


# speedup vs baseline: 9647.9346x; 1.4055x over previous
import functools

import jax
import jax.numpy as jnp
from jax.experimental import pallas as pl
from jax.experimental.pallas import tpu as pltpu

N = 2048
TD = 256


def _layer_body(adj_ref, x_ref, w_ref, asrc_ref, adst_ref, b_ref, out_ref,
                hh_s, as_s, adt_s, *, heads, ch, self_loops, relu):
    j = pl.program_id(0)

    @pl.when(j == 0)
    def _proj():
        h = jnp.dot(x_ref[...], w_ref[...], preferred_element_type=jnp.float32)
        as_s[...] = jnp.dot(h, asrc_ref[...],
                            preferred_element_type=jnp.float32)
        adt_s[...] = jax.lax.dot_general(
            adst_ref[...], h, (((0,), (1,)), ((), ())),
            preferred_element_type=jnp.float32)
        for i in range(heads):
            hh_s[i * N:(i + 1) * N, 0:ch] = h[:, i * ch:(i + 1) * ch]
        hh_s[:, ch:ch + 1] = jnp.ones((heads * N, 1), jnp.float32)

    maskf = adj_ref[...]
    if self_loops:
        r = jax.lax.broadcasted_iota(jnp.int32, (N, TD), 0)
        c = jax.lax.broadcasted_iota(jnp.int32, (N, TD), 1) + j * TD
        maskf = jnp.where(r == c, 1.0, maskf)

    acc = jnp.zeros((TD, ch), jnp.float32)
    for i in range(heads):
        asv = as_s[:, i:i + 1]
        adv = adt_s[i:i + 1, pl.ds(j * TD, TD)]
        mp = jnp.max(asv) + adv
        mp = jnp.maximum(mp, mp * jnp.float32(0.2))
        e = asv + adv
        e = jnp.maximum(e, e * jnp.float32(0.2))
        p = jnp.exp(e - mp) * maskf
        res = jax.lax.dot_general(
            p, hh_s[i * N:(i + 1) * N, :], (((0,), (0,)), ((), ())),
            preferred_element_type=jnp.float32)
        acc = acc + res[:, :ch] / (res[:, ch:ch + 1] + jnp.float32(1e-16))
    out = acc * jnp.float32(1.0 / heads) + b_ref[0:1, :]
    if relu:
        out = jnp.maximum(out, 0.0)
    out_ref[...] = out


def _block_diag(a, ch):
    hc = a.shape[0] * ch
    onehot = ((jnp.arange(hc) // ch)[:, None] == jnp.arange(8)[None, :])
    return onehot.astype(jnp.float32) * a.reshape(-1, 1)


def _gat_layer(adj, x_in, W, a_src, a_dst, b, heads, ch, self_loops, relu):
    din = x_in.shape[1]
    return pl.pallas_call(
        functools.partial(_layer_body, heads=heads, ch=ch,
                          self_loops=self_loops, relu=relu),
        grid=(N // TD,),
        in_specs=[pl.BlockSpec((N, TD), lambda j: (0, j)),
                  pl.BlockSpec((N, din), lambda j: (0, 0)),
                  pl.BlockSpec((din, heads * ch), lambda j: (0, 0)),
                  pl.BlockSpec((heads * ch, 8), lambda j: (0, 0)),
                  pl.BlockSpec((heads * ch, 8), lambda j: (0, 0)),
                  pl.BlockSpec((1, ch), lambda j: (0, 0))],
        out_specs=pl.BlockSpec((TD, ch), lambda j: (j, 0)),
        out_shape=jax.ShapeDtypeStruct((N, ch), jnp.float32),
        scratch_shapes=[pltpu.VMEM((heads * N, ch + 1), jnp.float32),
                        pltpu.VMEM((N, 8), jnp.float32),
                        pltpu.VMEM((8, N), jnp.float32)],
    )(adj, x_in, W, _block_diag(a_src, ch), _block_diag(a_dst, ch),
      b.reshape(1, ch))


def kernel(adj, x, W0, a_src0, a_dst0, b0, W1, a_src1, a_dst1, b1,
           W2, a_src2, a_dst2, b2):
    h = _gat_layer(adj, x, W0, a_src0, a_dst0, b0, 4, 16, False, True)
    h = _gat_layer(adj, h, W1, a_src1, a_dst1, b1, 4, 16, True, True)
    return _gat_layer(adj, h, W2, a_src2, a_dst2, b2, 1, 32, True, False)


# speedup vs baseline: 12219.9056x; 1.2666x over previous
import functools

import jax
import jax.numpy as jnp
from jax.experimental import pallas as pl
from jax.experimental.pallas import tpu as pltpu

N = 2048
TD = 256


def _layer_body(adj_ref, x_ref, w_ref, asrc_ref, adst_ref, b_ref, out_ref,
                *rest, heads, ch, mode, relu):
    if mode == 1:
        adjd_ref, hh_s, as_s, adt_s, am_s = rest
    else:
        hh_s, as_s, adt_s, am_s = rest
    j = pl.program_id(0)

    @pl.when(j == 0)
    def _proj():
        h = jnp.dot(x_ref[...], w_ref[...], preferred_element_type=jnp.float32)
        a_s = jnp.dot(h, asrc_ref[...], preferred_element_type=jnp.float32)
        as_s[...] = a_s.astype(jnp.bfloat16)
        am_s[...] = jnp.max(a_s, axis=0, keepdims=True).astype(jnp.bfloat16)
        adt_s[...] = jax.lax.dot_general(
            adst_ref[...], h, (((0,), (1,)), ((), ())),
            preferred_element_type=jnp.float32).astype(jnp.bfloat16)
        for i in range(heads):
            hh_s[i * N:(i + 1) * N, 0:ch] = (
                h[:, i * ch:(i + 1) * ch].astype(jnp.bfloat16))
        hh_s[:, ch:ch + 1] = jnp.ones((heads * N, 1), jnp.bfloat16)

    maskf = adj_ref[...]
    if mode == 1:
        r = jax.lax.broadcasted_iota(jnp.int32, (N, TD), 0)
        c = jax.lax.broadcasted_iota(jnp.int32, (N, TD), 1) + j * TD
        maskf = jnp.where(r == c, jnp.bfloat16(1), maskf)
        adjd_ref[...] = maskf

    slope = jnp.bfloat16(0.2)
    acc = jnp.zeros((TD, ch), jnp.float32)
    for i in range(heads):
        asv = as_s[:, i:i + 1]
        adv = adt_s[i:i + 1, pl.ds(j * TD, TD)]
        mp = am_s[0:1, i:i + 1] + adv
        mp = jnp.maximum(mp, mp * slope)
        e = asv + adv
        e = jnp.maximum(e, e * slope)
        p = jnp.exp(e - mp) * maskf
        res = jax.lax.dot_general(
            p, hh_s[i * N:(i + 1) * N, :], (((0,), (0,)), ((), ())),
            preferred_element_type=jnp.float32)
        acc = acc + res[:, :ch] / (res[:, ch:ch + 1] + jnp.float32(1e-16))
    out = acc * jnp.float32(1.0 / heads) + b_ref[0:1, :]
    if relu:
        out = jnp.maximum(out, 0.0)
    out_ref[...] = out


def _block_diag(a, ch):
    hc = a.shape[0] * ch
    onehot = ((jnp.arange(hc) // ch)[:, None] == jnp.arange(8)[None, :])
    return onehot.astype(jnp.float32) * a.reshape(-1, 1)


def _gat_layer(mask, x_in, W, a_src, a_dst, b, heads, ch, mode, relu):
    din = x_in.shape[1]
    out_shape = jax.ShapeDtypeStruct((N, ch), jnp.float32)
    out_spec = pl.BlockSpec((TD, ch), lambda j: (j, 0))
    if mode == 1:
        out_shape = (out_shape,
                     jax.ShapeDtypeStruct((N, N), jnp.bfloat16))
        out_spec = (out_spec, pl.BlockSpec((N, TD), lambda j: (0, j)))
    return pl.pallas_call(
        functools.partial(_layer_body, heads=heads, ch=ch,
                          mode=mode, relu=relu),
        grid=(N // TD,),
        in_specs=[pl.BlockSpec((N, TD), lambda j: (0, j)),
                  pl.BlockSpec((N, din), lambda j: (0, 0)),
                  pl.BlockSpec((din, heads * ch), lambda j: (0, 0)),
                  pl.BlockSpec((heads * ch, 8), lambda j: (0, 0)),
                  pl.BlockSpec((heads * ch, 8), lambda j: (0, 0)),
                  pl.BlockSpec((1, ch), lambda j: (0, 0))],
        out_specs=out_spec,
        out_shape=out_shape,
        scratch_shapes=[pltpu.VMEM((heads * N, ch + 1), jnp.bfloat16),
                        pltpu.VMEM((N, 8), jnp.bfloat16),
                        pltpu.VMEM((8, N), jnp.bfloat16),
                        pltpu.VMEM((1, 8), jnp.bfloat16)],
    )(mask, x_in, W, _block_diag(a_src, ch), _block_diag(a_dst, ch),
      b.reshape(1, ch))


def kernel(adj, x, W0, a_src0, a_dst0, b0, W1, a_src1, a_dst1, b1,
           W2, a_src2, a_dst2, b2):
    adj16 = adj.astype(jnp.bfloat16)
    h = _gat_layer(adj16, x, W0, a_src0, a_dst0, b0, 4, 16, 0, True)
    h, adjd = _gat_layer(adj16, h, W1, a_src1, a_dst1, b1, 4, 16, 1, True)
    return _gat_layer(adjd, h, W2, a_src2, a_dst2, b2, 1, 32, 2, False)


# speedup vs baseline: 15102.3453x; 1.2359x over previous
import functools

import jax
import jax.numpy as jnp
from jax.experimental import pallas as pl
from jax.experimental.pallas import tpu as pltpu

N = 2048
TD = 256


def _layer_body(adj_ref, x_ref, w_ref, asrc_ref, adst_ref, b_ref, out_ref,
                *rest, heads, ch, mode, relu, x_t):
    if mode == 1:
        adjd_ref, hh_s, as_s, adt_s, am_s = rest
    else:
        hh_s, as_s, adt_s, am_s = rest
    j = pl.program_id(0)

    @pl.when(j == 0)
    def _proj():
        if x_t:
            h = jax.lax.dot_general(
                x_ref[...], w_ref[...], (((0,), (0,)), ((), ())),
                preferred_element_type=jnp.float32)
        else:
            h = jnp.dot(x_ref[...], w_ref[...],
                        preferred_element_type=jnp.float32)
        a_s = jnp.dot(h, asrc_ref[...], preferred_element_type=jnp.float32)
        as_s[...] = a_s.astype(jnp.bfloat16)
        am_s[...] = jnp.max(a_s, axis=0, keepdims=True).astype(jnp.bfloat16)
        adt_s[...] = jax.lax.dot_general(
            adst_ref[...], h, (((0,), (1,)), ((), ())),
            preferred_element_type=jnp.float32).astype(jnp.bfloat16)
        for i in range(heads):
            hh_s[i * N:(i + 1) * N, 0:ch] = (
                h[:, i * ch:(i + 1) * ch].astype(jnp.bfloat16))
        hh_s[:, ch:ch + 1] = jnp.ones((heads * N, 1), jnp.bfloat16)

    maskf = adj_ref[...]
    if mode == 1:
        r = jax.lax.broadcasted_iota(jnp.int32, (N, TD), 0)
        c = jax.lax.broadcasted_iota(jnp.int32, (N, TD), 1) + j * TD
        maskf = jnp.where(r == c, jnp.bfloat16(1), maskf)
        adjd_ref[...] = maskf

    slope = jnp.bfloat16(0.2)
    acc = jnp.zeros((ch, TD), jnp.float32)
    for i in range(heads):
        asv = as_s[:, i:i + 1]
        adv = adt_s[i:i + 1, pl.ds(j * TD, TD)]
        mp = am_s[0:1, i:i + 1] + adv
        mp = jnp.maximum(mp, mp * slope)
        e = asv + adv
        e = jnp.maximum(e, e * slope)
        p = jnp.exp(e - mp) * maskf
        res = jax.lax.dot_general(
            hh_s[i * N:(i + 1) * N, :], p, (((0,), (0,)), ((), ())),
            preferred_element_type=jnp.float32)
        acc = acc + res[:ch, :] / (res[ch:ch + 1, :] + jnp.float32(1e-16))
    out = acc * jnp.float32(1.0 / heads) + b_ref[:, 0:1]
    if relu:
        out = jnp.maximum(out, 0.0)
    out_ref[...] = out


def _block_diag(a, ch):
    hc = a.shape[0] * ch
    onehot = ((jnp.arange(hc) // ch)[:, None] == jnp.arange(8)[None, :])
    return onehot.astype(jnp.float32) * a.reshape(-1, 1)


def _gat_layer(mask, x_in, W, a_src, a_dst, b, heads, ch, mode, relu, x_t):
    xspec = pl.BlockSpec(x_in.shape, lambda j: (0, 0))
    out_shape = jax.ShapeDtypeStruct((ch, N), jnp.float32)
    out_spec = pl.BlockSpec((ch, TD), lambda j: (0, j))
    if mode == 1:
        out_shape = (out_shape,
                     jax.ShapeDtypeStruct((N, N), jnp.bfloat16))
        out_spec = (out_spec, pl.BlockSpec((N, TD), lambda j: (0, j)))
    return pl.pallas_call(
        functools.partial(_layer_body, heads=heads, ch=ch,
                          mode=mode, relu=relu, x_t=x_t),
        grid=(N // TD,),
        in_specs=[pl.BlockSpec((N, TD), lambda j: (0, j)),
                  xspec,
                  pl.BlockSpec(W.shape, lambda j: (0, 0)),
                  pl.BlockSpec((heads * ch, 8), lambda j: (0, 0)),
                  pl.BlockSpec((heads * ch, 8), lambda j: (0, 0)),
                  pl.BlockSpec((ch, 1), lambda j: (0, 0))],
        out_specs=out_spec,
        out_shape=out_shape,
        scratch_shapes=[pltpu.VMEM((heads * N, ch + 1), jnp.bfloat16),
                        pltpu.VMEM((N, 8), jnp.bfloat16),
                        pltpu.VMEM((8, N), jnp.bfloat16),
                        pltpu.VMEM((1, 8), jnp.bfloat16)],
    )(mask, x_in, W, _block_diag(a_src, ch), _block_diag(a_dst, ch),
      b.reshape(ch, 1))


def kernel(adj, x, W0, a_src0, a_dst0, b0, W1, a_src1, a_dst1, b1,
           W2, a_src2, a_dst2, b2):
    adj16 = adj.astype(jnp.bfloat16)
    h = _gat_layer(adj16, x, W0, a_src0, a_dst0, b0, 4, 16, 0, True, False)
    h, adjd = _gat_layer(adj16, h, W1, a_src1, a_dst1, b1, 4, 16, 1, True,
                         True)
    out_t = _gat_layer(adjd, h, W2, a_src2, a_dst2, b2, 1, 32, 2, False, True)
    return out_t.T


# speedup vs baseline: 17284.4884x; 1.1445x over previous
import jax
import jax.numpy as jnp
from jax.experimental import pallas as pl
from jax.experimental.pallas import tpu as pltpu

N = 2048
TD = 256
NT = N // TD


def _mega_body(adj_ref, x_ref, w0_ref, s0_ref, d0_ref, b0_ref,
               w1_ref, s1_ref, d1_ref, b1_ref,
               w2_ref, s2_ref, d2_ref, b2_ref, out_ref,
               adjd_s, hh_s, as_s, adt_s, am_s, xa_s):
    j = pl.program_id(0)
    t = jax.lax.rem(j, NT)
    col0 = t * TD

    def proj(h, asrc_ref, adst_ref, heads, ch):
        a_s = jnp.dot(h, asrc_ref[...], preferred_element_type=jnp.float32)
        as_s[...] = a_s.astype(jnp.bfloat16)
        am_s[...] = jnp.max(a_s, axis=0, keepdims=True).astype(jnp.bfloat16)
        adt_s[...] = jax.lax.dot_general(
            adst_ref[...], h, (((0,), (1,)), ((), ())),
            preferred_element_type=jnp.float32).astype(jnp.bfloat16)
        for i in range(heads):
            hh_s[i * N:(i + 1) * N, 0:ch] = (
                h[:, i * ch:(i + 1) * ch].astype(jnp.bfloat16))
        hh_s[0:heads * N, ch:ch + 1] = jnp.ones((heads * N, 1), jnp.bfloat16)

    @pl.when(j == 0)
    def _proj0():
        h = jnp.dot(x_ref[...], w0_ref[...],
                    preferred_element_type=jnp.float32)
        proj(h, s0_ref, d0_ref, 4, 16)

    @pl.when(j == NT)
    def _proj1():
        h = jax.lax.dot_general(
            xa_s[...], w1_ref[...], (((0,), (0,)), ((), ())),
            preferred_element_type=jnp.float32)
        proj(h, s1_ref, d1_ref, 4, 16)

    @pl.when(j == 2 * NT)
    def _proj2():
        h = jax.lax.dot_general(
            xa_s[...], w2_ref[...], (((0,), (0,)), ((), ())),
            preferred_element_type=jnp.float32)
        proj(h, s2_ref, d2_ref, 1, 32)

    slope = jnp.bfloat16(0.2)

    def attn(maskf, heads, ch):
        acc = jnp.zeros((ch, TD), jnp.float32)
        for i in range(heads):
            asv = as_s[:, i:i + 1]
            adv = adt_s[i:i + 1, pl.ds(col0, TD)]
            mp = am_s[0:1, i:i + 1] + adv
            mp = jnp.maximum(mp, mp * slope)
            e = asv + adv
            e = jnp.maximum(e, e * slope)
            p = jnp.exp(e - mp) * maskf
            res = jax.lax.dot_general(
                hh_s[i * N:(i + 1) * N, 0:ch + 1], p,
                (((0,), (0,)), ((), ())),
                preferred_element_type=jnp.float32)
            acc = acc + res[:ch, :] / (res[ch:ch + 1, :] +
                                       jnp.float32(1e-16))
        return acc * jnp.float32(1.0 / heads)

    @pl.when(j < NT)
    def _layer0():
        maskf = adj_ref[...].astype(jnp.bfloat16)
        r = jax.lax.broadcasted_iota(jnp.int32, (N, TD), 0)
        c = jax.lax.broadcasted_iota(jnp.int32, (N, TD), 1) + col0
        adjd_s[:, pl.ds(col0, TD)] = jnp.where(r == c, jnp.bfloat16(1), maskf)
        out = attn(maskf, 4, 16) + b0_ref[:, 0:1]
        xa_s[:, pl.ds(col0, TD)] = jnp.maximum(out, 0.0)

    @pl.when((j >= NT) & (j < 2 * NT))
    def _layer1():
        maskf = adjd_s[:, pl.ds(col0, TD)]
        out = attn(maskf, 4, 16) + b1_ref[:, 0:1]
        xa_s[:, pl.ds(col0, TD)] = jnp.maximum(out, 0.0)

    @pl.when(j >= 2 * NT)
    def _layer2():
        maskf = adjd_s[:, pl.ds(col0, TD)]
        out_ref[...] = attn(maskf, 1, 32) + b2_ref[:, 0:1]


def _block_diag(a, ch):
    hc = a.shape[0] * ch
    onehot = ((jnp.arange(hc) // ch)[:, None] == jnp.arange(8)[None, :])
    return onehot.astype(jnp.float32) * a.reshape(-1, 1)


def _const(shape):
    return pl.BlockSpec(shape, lambda j: (0, 0))


def kernel(adj, x, W0, a_src0, a_dst0, b0, W1, a_src1, a_dst1, b1,
           W2, a_src2, a_dst2, b2):
    out_t = pl.pallas_call(
        _mega_body,
        grid=(3 * NT,),
        in_specs=[
            pl.BlockSpec((N, TD), lambda j: (0, jnp.minimum(j, NT - 1))),
            _const((N, 128)),
            _const((128, 64)), _const((64, 8)), _const((64, 8)),
            _const((16, 1)),
            _const((16, 64)), _const((64, 8)), _const((64, 8)),
            _const((16, 1)),
            _const((16, 32)), _const((32, 8)), _const((32, 8)),
            _const((32, 1)),
        ],
        out_specs=pl.BlockSpec((32, TD),
                               lambda j: (0, jnp.maximum(j - 2 * NT, 0))),
        out_shape=jax.ShapeDtypeStruct((32, N), jnp.float32),
        scratch_shapes=[pltpu.VMEM((N, N), jnp.bfloat16),
                        pltpu.VMEM((4 * N, 33), jnp.bfloat16),
                        pltpu.VMEM((N, 8), jnp.bfloat16),
                        pltpu.VMEM((8, N), jnp.bfloat16),
                        pltpu.VMEM((1, 8), jnp.bfloat16),
                        pltpu.VMEM((16, N), jnp.float32)],
    )(adj, x,
      W0, _block_diag(a_src0, 16), _block_diag(a_dst0, 16), b0.reshape(16, 1),
      W1, _block_diag(a_src1, 16), _block_diag(a_dst1, 16), b1.reshape(16, 1),
      W2, _block_diag(a_src2, 32), _block_diag(a_dst2, 32), b2.reshape(32, 1))
    return out_t.T


# speedup vs baseline: 19864.2216x; 1.1493x over previous
import jax
import jax.numpy as jnp
from jax.experimental import pallas as pl
from jax.experimental.pallas import tpu as pltpu

N = 2048
TD = 512
NT = N // TD


def _mega_body(adj_ref, x_ref, w0_ref, s0_ref, d0_ref, b0_ref,
               w1_ref, s1_ref, d1_ref, b1_ref,
               w2_ref, s2_ref, d2_ref, b2_ref, out_ref,
               adjd_s, hh_s, as_s, adt_s, am_s, xa_s):
    j = pl.program_id(0)
    t = jax.lax.rem(j, NT)
    col0 = t * TD

    def proj(h, asrc_ref, adst_ref, heads, ch):
        a_s = jnp.dot(h, asrc_ref[...], preferred_element_type=jnp.float32)
        as_s[...] = a_s.astype(jnp.bfloat16)
        am_s[...] = jnp.max(a_s, axis=0, keepdims=True).astype(jnp.bfloat16)
        adt_s[...] = jax.lax.dot_general(
            adst_ref[...], h, (((0,), (1,)), ((), ())),
            preferred_element_type=jnp.float32).astype(jnp.bfloat16)
        for i in range(heads):
            hh_s[i * N:(i + 1) * N, 0:ch] = (
                h[:, i * ch:(i + 1) * ch].astype(jnp.bfloat16))
        hh_s[0:heads * N, ch:ch + 1] = jnp.ones((heads * N, 1), jnp.bfloat16)

    @pl.when(j == 0)
    def _proj0():
        h = jnp.dot(x_ref[...], w0_ref[...],
                    preferred_element_type=jnp.float32)
        proj(h, s0_ref, d0_ref, 4, 16)

    @pl.when(j == NT)
    def _proj1():
        h = jax.lax.dot_general(
            xa_s[...], w1_ref[...], (((0,), (0,)), ((), ())),
            preferred_element_type=jnp.float32)
        proj(h, s1_ref, d1_ref, 4, 16)

    @pl.when(j == 2 * NT)
    def _proj2():
        h = jax.lax.dot_general(
            xa_s[...], w2_ref[...], (((0,), (0,)), ((), ())),
            preferred_element_type=jnp.float32)
        proj(h, s2_ref, d2_ref, 1, 32)

    slope = jnp.bfloat16(0.2)

    def attn(maskf, heads, ch):
        acc = jnp.zeros((ch, TD), jnp.float32)
        for i in range(heads):
            asv = as_s[:, i:i + 1]
            adv = adt_s[i:i + 1, pl.ds(col0, TD)]
            mp = am_s[0:1, i:i + 1] + adv
            mp = jnp.maximum(mp, mp * slope)
            e = asv + adv
            e = jnp.maximum(e, e * slope)
            p = jnp.exp(e - mp) * maskf
            res = jax.lax.dot_general(
                hh_s[i * N:(i + 1) * N, 0:ch + 1], p,
                (((0,), (0,)), ((), ())),
                preferred_element_type=jnp.float32)
            acc = acc + res[:ch, :] / (res[ch:ch + 1, :] +
                                       jnp.float32(1e-16))
        return acc * jnp.float32(1.0 / heads)

    @pl.when(j < NT)
    def _layer0():
        maskf = adj_ref[...].astype(jnp.bfloat16)
        r = jax.lax.broadcasted_iota(jnp.int32, (N, TD), 0)
        c = jax.lax.broadcasted_iota(jnp.int32, (N, TD), 1) + col0
        adjd_s[:, pl.ds(col0, TD)] = jnp.where(r == c, jnp.bfloat16(1), maskf)
        out = attn(maskf, 4, 16) + b0_ref[:, 0:1]
        xa_s[:, pl.ds(col0, TD)] = jnp.maximum(out, 0.0)

    @pl.when((j >= NT) & (j < 2 * NT))
    def _layer1():
        maskf = adjd_s[:, pl.ds(col0, TD)]
        out = attn(maskf, 4, 16) + b1_ref[:, 0:1]
        xa_s[:, pl.ds(col0, TD)] = jnp.maximum(out, 0.0)

    @pl.when(j >= 2 * NT)
    def _layer2():
        maskf = adjd_s[:, pl.ds(col0, TD)]
        out_ref[...] = attn(maskf, 1, 32) + b2_ref[:, 0:1]


def _block_diag(a, ch):
    hc = a.shape[0] * ch
    onehot = ((jnp.arange(hc) // ch)[:, None] == jnp.arange(8)[None, :])
    return onehot.astype(jnp.float32) * a.reshape(-1, 1)


def _const(shape):
    return pl.BlockSpec(shape, lambda j: (0, 0))


def kernel(adj, x, W0, a_src0, a_dst0, b0, W1, a_src1, a_dst1, b1,
           W2, a_src2, a_dst2, b2):
    out_t = pl.pallas_call(
        _mega_body,
        grid=(3 * NT,),
        in_specs=[
            pl.BlockSpec((N, TD), lambda j: (0, jnp.minimum(j, NT - 1))),
            _const((N, 128)),
            _const((128, 64)), _const((64, 8)), _const((64, 8)),
            _const((16, 1)),
            _const((16, 64)), _const((64, 8)), _const((64, 8)),
            _const((16, 1)),
            _const((16, 32)), _const((32, 8)), _const((32, 8)),
            _const((32, 1)),
        ],
        out_specs=pl.BlockSpec((32, TD),
                               lambda j: (0, jnp.maximum(j - 2 * NT, 0))),
        out_shape=jax.ShapeDtypeStruct((32, N), jnp.float32),
        scratch_shapes=[pltpu.VMEM((N, N), jnp.bfloat16),
                        pltpu.VMEM((4 * N, 33), jnp.bfloat16),
                        pltpu.VMEM((N, 8), jnp.bfloat16),
                        pltpu.VMEM((8, N), jnp.bfloat16),
                        pltpu.VMEM((1, 8), jnp.bfloat16),
                        pltpu.VMEM((16, N), jnp.float32)],
    )(adj, x,
      W0, _block_diag(a_src0, 16), _block_diag(a_dst0, 16), b0.reshape(16, 1),
      W1, _block_diag(a_src1, 16), _block_diag(a_dst1, 16), b1.reshape(16, 1),
      W2, _block_diag(a_src2, 32), _block_diag(a_dst2, 32), b2.reshape(32, 1))
    return out_t.T


# speedup vs baseline: 20567.9275x; 1.0354x over previous
import jax
import jax.numpy as jnp
from jax.experimental import pallas as pl
from jax.experimental.pallas import tpu as pltpu

N = 2048
TD = 1024
NT = N // TD


def _mega_body(adj_ref, x_ref, w0_ref, s0_ref, d0_ref, b0_ref,
               w1_ref, s1_ref, d1_ref, b1_ref,
               w2_ref, s2_ref, d2_ref, b2_ref, out_ref,
               adjd_s, hh_s, as_s, adt_s, am_s, xa_s):
    j = pl.program_id(0)
    t = jax.lax.rem(j, NT)
    col0 = t * TD

    def proj(h, asrc_ref, adst_ref, heads, ch):
        a_s = jnp.dot(h, asrc_ref[...], preferred_element_type=jnp.float32)
        as_s[...] = a_s.astype(jnp.bfloat16)
        am_s[...] = jnp.max(a_s, axis=0, keepdims=True).astype(jnp.bfloat16)
        adt_s[...] = jax.lax.dot_general(
            adst_ref[...], h, (((0,), (1,)), ((), ())),
            preferred_element_type=jnp.float32).astype(jnp.bfloat16)
        for i in range(heads):
            hh_s[i * N:(i + 1) * N, 0:ch] = (
                h[:, i * ch:(i + 1) * ch].astype(jnp.bfloat16))
        hh_s[0:heads * N, ch:ch + 1] = jnp.ones((heads * N, 1), jnp.bfloat16)

    @pl.when(j == 0)
    def _proj0():
        h = jnp.dot(x_ref[...], w0_ref[...],
                    preferred_element_type=jnp.float32)
        proj(h, s0_ref, d0_ref, 4, 16)

    @pl.when(j == NT)
    def _proj1():
        h = jax.lax.dot_general(
            xa_s[...], w1_ref[...], (((0,), (0,)), ((), ())),
            preferred_element_type=jnp.float32)
        proj(h, s1_ref, d1_ref, 4, 16)

    @pl.when(j == 2 * NT)
    def _proj2():
        h = jax.lax.dot_general(
            xa_s[...], w2_ref[...], (((0,), (0,)), ((), ())),
            preferred_element_type=jnp.float32)
        proj(h, s2_ref, d2_ref, 1, 32)

    slope = jnp.bfloat16(0.2)

    def attn(maskf, heads, ch):
        acc = jnp.zeros((ch, TD), jnp.float32)
        for i in range(heads):
            asv = as_s[:, i:i + 1]
            adv = adt_s[i:i + 1, pl.ds(col0, TD)]
            mp = am_s[0:1, i:i + 1] + adv
            mp = jnp.maximum(mp, mp * slope)
            e = asv + adv
            e = jnp.maximum(e, e * slope)
            p = jnp.exp(e - mp) * maskf
            res = jax.lax.dot_general(
                hh_s[i * N:(i + 1) * N, 0:ch + 1], p,
                (((0,), (0,)), ((), ())),
                preferred_element_type=jnp.float32)
            acc = acc + res[:ch, :] / (res[ch:ch + 1, :] +
                                       jnp.float32(1e-16))
        return acc * jnp.float32(1.0 / heads)

    @pl.when(j < NT)
    def _layer0():
        maskf = adj_ref[...].astype(jnp.bfloat16)
        r = jax.lax.broadcasted_iota(jnp.int32, (N, TD), 0)
        c = jax.lax.broadcasted_iota(jnp.int32, (N, TD), 1) + col0
        adjd_s[:, pl.ds(col0, TD)] = jnp.where(r == c, jnp.bfloat16(1), maskf)
        out = attn(maskf, 4, 16) + b0_ref[:, 0:1]
        xa_s[:, pl.ds(col0, TD)] = jnp.maximum(out, 0.0)

    @pl.when((j >= NT) & (j < 2 * NT))
    def _layer1():
        maskf = adjd_s[:, pl.ds(col0, TD)]
        out = attn(maskf, 4, 16) + b1_ref[:, 0:1]
        xa_s[:, pl.ds(col0, TD)] = jnp.maximum(out, 0.0)

    @pl.when(j >= 2 * NT)
    def _layer2():
        maskf = adjd_s[:, pl.ds(col0, TD)]
        out_ref[...] = attn(maskf, 1, 32) + b2_ref[:, 0:1]


def _block_diag(a, ch):
    hc = a.shape[0] * ch
    onehot = ((jnp.arange(hc) // ch)[:, None] == jnp.arange(8)[None, :])
    return onehot.astype(jnp.float32) * a.reshape(-1, 1)


def _const(shape):
    return pl.BlockSpec(shape, lambda j: (0, 0))


def kernel(adj, x, W0, a_src0, a_dst0, b0, W1, a_src1, a_dst1, b1,
           W2, a_src2, a_dst2, b2):
    out_t = pl.pallas_call(
        _mega_body,
        grid=(3 * NT,),
        in_specs=[
            pl.BlockSpec((N, TD), lambda j: (0, jnp.minimum(j, NT - 1))),
            _const((N, 128)),
            _const((128, 64)), _const((64, 8)), _const((64, 8)),
            _const((16, 1)),
            _const((16, 64)), _const((64, 8)), _const((64, 8)),
            _const((16, 1)),
            _const((16, 32)), _const((32, 8)), _const((32, 8)),
            _const((32, 1)),
        ],
        out_specs=pl.BlockSpec((32, TD),
                               lambda j: (0, jnp.maximum(j - 2 * NT, 0))),
        out_shape=jax.ShapeDtypeStruct((32, N), jnp.float32),
        scratch_shapes=[pltpu.VMEM((N, N), jnp.bfloat16),
                        pltpu.VMEM((4 * N, 33), jnp.bfloat16),
                        pltpu.VMEM((N, 8), jnp.bfloat16),
                        pltpu.VMEM((8, N), jnp.bfloat16),
                        pltpu.VMEM((1, 8), jnp.bfloat16),
                        pltpu.VMEM((16, N), jnp.float32)],
    )(adj, x,
      W0, _block_diag(a_src0, 16), _block_diag(a_dst0, 16), b0.reshape(16, 1),
      W1, _block_diag(a_src1, 16), _block_diag(a_dst1, 16), b1.reshape(16, 1),
      W2, _block_diag(a_src2, 32), _block_diag(a_dst2, 32), b2.reshape(32, 1))
    return out_t.T


# speedup vs baseline: 21694.2524x; 1.0548x over previous
import jax
import jax.numpy as jnp
from jax.experimental import pallas as pl
from jax.experimental.pallas import tpu as pltpu

N = 2048
TD = 2048
NT = N // TD


def _mega_body(adj_ref, x_ref, w0_ref, s0_ref, d0_ref, b0_ref,
               w1_ref, s1_ref, d1_ref, b1_ref,
               w2_ref, s2_ref, d2_ref, b2_ref, out_ref,
               adjd_s, hh_s, as_s, adt_s, am_s, xa_s):
    j = pl.program_id(0)
    t = jax.lax.rem(j, NT)
    col0 = t * TD

    def proj(h, asrc_ref, adst_ref, heads, ch):
        a_s = jnp.dot(h, asrc_ref[...], preferred_element_type=jnp.float32)
        as_s[...] = a_s.astype(jnp.bfloat16)
        am_s[...] = jnp.max(a_s, axis=0, keepdims=True).astype(jnp.bfloat16)
        adt_s[...] = jax.lax.dot_general(
            adst_ref[...], h, (((0,), (1,)), ((), ())),
            preferred_element_type=jnp.float32).astype(jnp.bfloat16)
        for i in range(heads):
            hh_s[i * N:(i + 1) * N, 0:ch] = (
                h[:, i * ch:(i + 1) * ch].astype(jnp.bfloat16))
        hh_s[0:heads * N, ch:ch + 1] = jnp.ones((heads * N, 1), jnp.bfloat16)

    @pl.when(j == 0)
    def _proj0():
        h = jnp.dot(x_ref[...], w0_ref[...],
                    preferred_element_type=jnp.float32)
        proj(h, s0_ref, d0_ref, 4, 16)

    @pl.when(j == NT)
    def _proj1():
        h = jax.lax.dot_general(
            xa_s[...], w1_ref[...], (((0,), (0,)), ((), ())),
            preferred_element_type=jnp.float32)
        proj(h, s1_ref, d1_ref, 4, 16)

    @pl.when(j == 2 * NT)
    def _proj2():
        h = jax.lax.dot_general(
            xa_s[...], w2_ref[...], (((0,), (0,)), ((), ())),
            preferred_element_type=jnp.float32)
        proj(h, s2_ref, d2_ref, 1, 32)

    slope = jnp.bfloat16(0.2)

    def attn(maskf, heads, ch):
        acc = jnp.zeros((ch, TD), jnp.float32)
        for i in range(heads):
            asv = as_s[:, i:i + 1]
            adv = adt_s[i:i + 1, pl.ds(col0, TD)]
            mp = am_s[0:1, i:i + 1] + adv
            mp = jnp.maximum(mp, mp * slope)
            e = asv + adv
            e = jnp.maximum(e, e * slope)
            p = jnp.exp(e - mp) * maskf
            res = jax.lax.dot_general(
                hh_s[i * N:(i + 1) * N, 0:ch + 1], p,
                (((0,), (0,)), ((), ())),
                preferred_element_type=jnp.float32)
            acc = acc + res[:ch, :] / (res[ch:ch + 1, :] +
                                       jnp.float32(1e-16))
        return acc * jnp.float32(1.0 / heads)

    @pl.when(j < NT)
    def _layer0():
        maskf = adj_ref[...].astype(jnp.bfloat16)
        r = jax.lax.broadcasted_iota(jnp.int32, (N, TD), 0)
        c = jax.lax.broadcasted_iota(jnp.int32, (N, TD), 1) + col0
        adjd_s[:, pl.ds(col0, TD)] = jnp.where(r == c, jnp.bfloat16(1), maskf)
        out = attn(maskf, 4, 16) + b0_ref[:, 0:1]
        xa_s[:, pl.ds(col0, TD)] = jnp.maximum(out, 0.0)

    @pl.when((j >= NT) & (j < 2 * NT))
    def _layer1():
        maskf = adjd_s[:, pl.ds(col0, TD)]
        out = attn(maskf, 4, 16) + b1_ref[:, 0:1]
        xa_s[:, pl.ds(col0, TD)] = jnp.maximum(out, 0.0)

    @pl.when(j >= 2 * NT)
    def _layer2():
        maskf = adjd_s[:, pl.ds(col0, TD)]
        out_ref[...] = attn(maskf, 1, 32) + b2_ref[:, 0:1]


def _block_diag(a, ch):
    hc = a.shape[0] * ch
    onehot = ((jnp.arange(hc) // ch)[:, None] == jnp.arange(8)[None, :])
    return onehot.astype(jnp.float32) * a.reshape(-1, 1)


def _const(shape):
    return pl.BlockSpec(shape, lambda j: (0, 0))


def kernel(adj, x, W0, a_src0, a_dst0, b0, W1, a_src1, a_dst1, b1,
           W2, a_src2, a_dst2, b2):
    out_t = pl.pallas_call(
        _mega_body,
        grid=(3 * NT,),
        in_specs=[
            pl.BlockSpec((N, TD), lambda j: (0, jnp.minimum(j, NT - 1))),
            _const((N, 128)),
            _const((128, 64)), _const((64, 8)), _const((64, 8)),
            _const((16, 1)),
            _const((16, 64)), _const((64, 8)), _const((64, 8)),
            _const((16, 1)),
            _const((16, 32)), _const((32, 8)), _const((32, 8)),
            _const((32, 1)),
        ],
        out_specs=pl.BlockSpec((32, TD),
                               lambda j: (0, jnp.maximum(j - 2 * NT, 0))),
        out_shape=jax.ShapeDtypeStruct((32, N), jnp.float32),
        scratch_shapes=[pltpu.VMEM((N, N), jnp.bfloat16),
                        pltpu.VMEM((4 * N, 33), jnp.bfloat16),
                        pltpu.VMEM((N, 8), jnp.bfloat16),
                        pltpu.VMEM((8, N), jnp.bfloat16),
                        pltpu.VMEM((1, 8), jnp.bfloat16),
                        pltpu.VMEM((16, N), jnp.float32)],
    )(adj, x,
      W0, _block_diag(a_src0, 16), _block_diag(a_dst0, 16), b0.reshape(16, 1),
      W1, _block_diag(a_src1, 16), _block_diag(a_dst1, 16), b1.reshape(16, 1),
      W2, _block_diag(a_src2, 32), _block_diag(a_dst2, 32), b2.reshape(32, 1))
    return out_t.T
